# Initial kernel scaffold; baseline (speedup 1.0000x reference)
import jax
import jax.numpy as jnp
from jax.experimental import pallas as pl


def kernel(w1t, b1, w2k, b2, w3t, b3, d1e, d2, x):
    raise NotImplementedError("write your pallas kernel here")


# speedup vs baseline: 4.1642x; 4.1642x over previous
import jax
import jax.numpy as jnp
from jax.experimental import pallas as pl
from jax.experimental.pallas import tpu as pltpu

TB = 8
C1, C2 = 16, 32
H1, W1 = 28, 28
WP1 = W1 + 4
HP1 = 34
L1 = HP1 * WP1
P1 = H1 * WP1
P1E = P1 + WP1 + 1
H2, W2 = 14, 14
WP2 = W2 + 4
HP2 = 20
L2 = HP2 * WP2
P2 = H2 * WP2
P2E = P2 + WP2 + 1
HO2, WO2 = 7, 7
FEAT = C2 * HO2 * WO2
NOUT = 10
BL = 512


def _conv_kernel(x_ref, w1b_ref, b1r_ref, d1e_ref, w2b_ref, b2r_ref, d2_ref,
                 o_ref):
    xb = x_ref[...]

    acc = jnp.dot(w1b_ref[0], xb[:, 0:P1E],
                  preferred_element_type=jnp.float32)
    for kh in range(5):
        for kw in range(5):
            t = kh * 5 + kw
            if t == 0:
                continue
            s = kh * WP1 + kw
            acc = acc + jnp.dot(w1b_ref[t], xb[:, s:s + P1E],
                                preferred_element_type=jnp.float32)
    vmax = jnp.maximum(
        jnp.maximum(acc[:, 0:P1], acc[:, 1:P1 + 1]),
        jnp.maximum(acc[:, WP1:WP1 + P1], acc[:, WP1 + 1:WP1 + 1 + P1]))
    h1 = jnp.maximum(vmax + b1r_ref[...], 0.0)
    h1p = jnp.dot(h1, d1e_ref[...],
                  preferred_element_type=jnp.float32)

    acc2 = jnp.dot(w2b_ref[0], h1p[:, 0:P2E],
                   preferred_element_type=jnp.float32)
    for kh in range(5):
        for kw in range(5):
            t = kh * 5 + kw
            if t == 0:
                continue
            s = kh * WP2 + kw
            acc2 = acc2 + jnp.dot(w2b_ref[t], h1p[:, s:s + P2E],
                                  preferred_element_type=jnp.float32)
    m2 = jnp.maximum(
        jnp.maximum(acc2[:, 0:P2], acc2[:, 1:P2 + 1]),
        jnp.maximum(acc2[:, WP2:WP2 + P2], acc2[:, WP2 + 1:WP2 + 1 + P2]))
    h2 = jnp.maximum(m2 + b2r_ref[...], 0.0)
    o_ref[...] = jnp.dot(h2, d2_ref[...],
                         preferred_element_type=jnp.float32)


def _linear_kernel(x_ref, w_ref, b_ref, o_ref):
    o_ref[...] = jnp.dot(x_ref[...], w_ref[...],
                         preferred_element_type=jnp.float32) + b_ref[...]


@jax.jit
def kernel(w1t, b1, w2k, b2, w3t, b3, d1e, d2, x):
    N = x.shape[0]
    npad = ((N + TB - 1) // TB) * TB
    if npad != N:
        x = jnp.pad(x, ((0, npad - N), (0, 0), (0, 0), (0, 0)))

    xp = jnp.pad(x, ((0, 0), (0, 0), (2, HP1 - 2 - H1), (2, WP1 - 2 - W1)))
    xp = xp.reshape(npad, L1)

    eye = jnp.eye(TB, dtype=jnp.float32)
    w1b = jnp.einsum('tc,nm->tncm', w1t[:, :, 0], eye).reshape(
        25, TB * C1, TB)
    w2r = w2k.reshape(C2, 25, C1).transpose(1, 0, 2)
    w2b = jnp.einsum('toc,nm->tnomc', w2r, eye).reshape(
        25, TB * C2, TB * C1)
    b1r = jnp.tile(b1, (TB, 1))
    b2r = jnp.tile(b2, (TB, 1))

    feats2 = pl.pallas_call(
        _conv_kernel,
        out_shape=jax.ShapeDtypeStruct((npad * C2, HO2 * WO2), jnp.float32),
        grid=(npad // TB,),
        in_specs=[
            pl.BlockSpec((TB, L1), lambda g: (g, 0)),
            pl.BlockSpec((25, TB * C1, TB), lambda g: (0, 0, 0)),
            pl.BlockSpec((TB * C1, 1), lambda g: (0, 0)),
            pl.BlockSpec((P1, L2), lambda g: (0, 0)),
            pl.BlockSpec((25, TB * C2, TB * C1), lambda g: (0, 0, 0)),
            pl.BlockSpec((TB * C2, 1), lambda g: (0, 0)),
            pl.BlockSpec((P2, HO2 * WO2), lambda g: (0, 0)),
        ],
        out_specs=pl.BlockSpec((TB * C2, HO2 * WO2), lambda g: (g, 0)),
        compiler_params=pltpu.CompilerParams(
            dimension_semantics=("parallel",)),
    )(xp, w1b, b1r, d1e, w2b, b2r, d2)

    flat = feats2.reshape(npad, FEAT)

    nlin = ((npad + BL - 1) // BL) * BL
    flat_l = flat if nlin == npad else jnp.pad(
        flat, ((0, nlin - npad), (0, 0)))
    logits = pl.pallas_call(
        _linear_kernel,
        out_shape=jax.ShapeDtypeStruct((nlin, NOUT), jnp.float32),
        grid=(nlin // BL,),
        in_specs=[
            pl.BlockSpec((BL, FEAT), lambda g: (g, 0)),
            pl.BlockSpec((FEAT, NOUT), lambda g: (0, 0)),
            pl.BlockSpec((1, NOUT), lambda g: (0, 0)),
        ],
        out_specs=pl.BlockSpec((BL, NOUT), lambda g: (g, 0)),
        compiler_params=pltpu.CompilerParams(
            dimension_semantics=("parallel",)),
    )(flat_l, w3t, b3)

    return logits[:N], flat[:N]
